# baseline (speedup 1.0000x reference)
import functools

import jax
import jax.numpy as jnp
from jax import lax
from jax.experimental import pallas as pl
from jax.experimental.pallas import tpu as pltpu
from jax.experimental.pallas import tpu_sc as plsc

T = 50
B = 128
K = 2048
NC = 2
NS = 16
NW = NC * NS
BPW = B // NW
LANES = 16
NACC = 4
GROUPS = K // (LANES * NACC)


def _gather_sc(lw, idx):
    mesh = plsc.VectorSubcoreMesh(core_axis_name="c", subcore_axis_name="s")

    @functools.partial(
        pl.kernel,
        out_type=jax.ShapeDtypeStruct((NW, LANES), jnp.float32),
        mesh=mesh,
        scratch_types=[
            pltpu.VMEM((BPW, K), jnp.float32),
            pltpu.VMEM((BPW, K), jnp.float32),
            pltpu.VMEM((BPW, K), jnp.float32),
            pltpu.VMEM((BPW, K), jnp.float32),
            pltpu.VMEM((BPW, K), jnp.int32),
            pltpu.VMEM((BPW, K), jnp.int32),
            pltpu.VMEM((BPW, K), jnp.int32),
            pltpu.VMEM((BPW, K), jnp.int32),
            pltpu.VMEM((LANES,), jnp.float32),
            pltpu.SemaphoreType.DMA,
            pltpu.SemaphoreType.DMA,
            pltpu.SemaphoreType.DMA,
            pltpu.SemaphoreType.DMA,
        ],
        compiler_params=pltpu.CompilerParams(needs_layout_passes=False),
    )
    def body(lw_hbm, idx_hbm, gat_hbm, lw0, lw1, lw2, lw3,
             idx0, idx1, idx2, idx3, out_v, sem0, sem1, sem2, sem3):
        wid = lax.axis_index("s") * NC + lax.axis_index("c")
        b0 = wid * BPW
        lane = lax.iota(jnp.int32, LANES)
        lw_bufs, idx_bufs = (lw0, lw1, lw2, lw3), (idx0, idx1, idx2, idx3)
        sems = (sem0, sem1, sem2, sem3)

        def issue(t, phase):
            src = pl.ds(b0, BPW)
            pltpu.async_copy(lw_hbm.at[t, src], lw_bufs[phase], sems[phase])
            pltpu.async_copy(idx_hbm.at[t, src], idx_bufs[phase], sems[phase])

        def drain(t, phase):
            src = pl.ds(b0, BPW)
            pltpu.make_async_copy(lw_hbm.at[t, src], lw_bufs[phase],
                                  sems[phase]).wait()
            pltpu.make_async_copy(idx_hbm.at[t, src], idx_bufs[phase],
                                  sems[phase]).wait()

        def compute(phase, accs):
            lw_v, idx_v = lw_bufs[phase], idx_bufs[phase]
            new_accs = []
            zero = jnp.zeros((LANES,), jnp.float32)
            for j in range(BPW):
                row = jnp.full((LANES,), j, jnp.int32)

                def group_body(i, carry, j=j, row=row):
                    acc = list(carry)
                    base = i * (LANES * NACC)
                    for a in range(NACC):
                        off = pl.multiple_of(base + a * LANES, LANES)
                        iv = idx_v[j, pl.ds(off, LANES)]
                        acc[a] = acc[a] + plsc.load_gather(lw_v, [row, iv])
                    return tuple(acc)

                init = (accs[j], zero, zero, zero)
                res = lax.fori_loop(0, GROUPS, group_body, init, unroll=2)
                new_accs.append((res[0] + res[1]) + (res[2] + res[3]))
            return tuple(new_accs)

        issue(0, 0)
        issue(1, 1)
        issue(2, 2)

        def quad_body(tp, accs):
            t = 4 * tp
            for ph in range(4):
                issue(jnp.minimum(t + ph + 3, T - 2), (ph + 3) % 4)
                drain(t + ph, ph)
                accs = compute(ph, accs)
            return accs

        zero = jnp.zeros((LANES,), jnp.float32)
        accs = lax.fori_loop(0, (T - 1) // 4, quad_body, (zero,) * BPW)
        drain(T - 2, 0)
        accs = compute(0, accs)
        drain(T - 2, 1)
        drain(T - 2, 2)

        out_vec = jnp.zeros((LANES,), jnp.float32)
        for j in range(BPW):
            out_vec = jnp.where(lane == j, jnp.sum(accs[j]), out_vec)
        out_v[...] = out_vec
        pltpu.sync_copy(out_v, gat_hbm.at[wid])

    return body(lw, idx)


def _lse_tc(lw):
    def body(lw_ref, out_ref):
        t = pl.program_id(0)
        x = lw_ref[0]
        m = jnp.max(x, axis=1, keepdims=True)
        s = jnp.sum(jnp.exp(x - m), axis=1)
        lse = m[:, 0] + jnp.log(s)

        @pl.when(t == 0)
        def _():
            out_ref[...] = jnp.zeros_like(out_ref)

        out_ref[0, :] += float(K) * lse

    return pl.pallas_call(
        body,
        grid=(T - 1,),
        in_specs=[pl.BlockSpec((1, B, K), lambda t: (t, 0, 0))],
        out_specs=pl.BlockSpec((1, B), lambda t: (0, 0)),
        out_shape=jax.ShapeDtypeStruct((1, B), jnp.float32),
    )(lw)


def kernel(log_weights, ancestral_indices):
    gat = _gather_sc(log_weights, ancestral_indices)
    lse = _lse_tc(log_weights)
    return gat[:, :BPW].reshape(B) - lse[0]


# speedup vs baseline: 1.0340x; 1.0340x over previous
import functools

import jax
import jax.numpy as jnp
from jax import lax
from jax.experimental import pallas as pl
from jax.experimental.pallas import tpu as pltpu
from jax.experimental.pallas import tpu_sc as plsc

T = 50
B = 128
K = 2048
NC = 2
NS = 16
NW = NC * NS
BPW = B // NW
LANES = 16
NACC = 4
GROUPS = K // (LANES * NACC)


def _gather_sc(lw, idx):
    mesh = plsc.VectorSubcoreMesh(core_axis_name="c", subcore_axis_name="s")

    @functools.partial(
        pl.kernel,
        out_type=jax.ShapeDtypeStruct((NW, LANES), jnp.float32),
        mesh=mesh,
        scratch_types=[
            pltpu.VMEM((BPW, K), jnp.float32),
            pltpu.VMEM((BPW, K), jnp.float32),
            pltpu.VMEM((BPW, K), jnp.float32),
            pltpu.VMEM((BPW, K), jnp.int32),
            pltpu.VMEM((BPW, K), jnp.int32),
            pltpu.VMEM((BPW, K), jnp.int32),
            pltpu.VMEM((LANES,), jnp.float32),
            pltpu.SemaphoreType.DMA,
            pltpu.SemaphoreType.DMA,
            pltpu.SemaphoreType.DMA,
        ],
        compiler_params=pltpu.CompilerParams(
            needs_layout_passes=False,
            skip_device_barrier=True,
            disable_bounds_checks=True,
            disable_semaphore_checks=True,
        ),
    )
    def body(lw_hbm, idx_hbm, gat_hbm, lw0, lw1, lw2, idx0, idx1, idx2,
             out_v, sem0, sem1, sem2):
        wid = lax.axis_index("s") * NC + lax.axis_index("c")
        b0 = wid * BPW
        lane = lax.iota(jnp.int32, LANES)
        lw_bufs, idx_bufs = (lw0, lw1, lw2), (idx0, idx1, idx2)
        sems = (sem0, sem1, sem2)

        def issue(t, phase):
            src = pl.ds(b0, BPW)
            pltpu.async_copy(lw_hbm.at[t, src], lw_bufs[phase], sems[phase])
            pltpu.async_copy(idx_hbm.at[t, src], idx_bufs[phase], sems[phase])

        def drain(t, phase):
            src = pl.ds(b0, BPW)
            pltpu.make_async_copy(lw_hbm.at[t, src], lw_bufs[phase],
                                  sems[phase]).wait()
            pltpu.make_async_copy(idx_hbm.at[t, src], idx_bufs[phase],
                                  sems[phase]).wait()

        def compute(phase, accs):
            lw_v, idx_v = lw_bufs[phase], idx_bufs[phase]
            new_accs = []
            zero = jnp.zeros((LANES,), jnp.float32)
            for j in range(BPW):
                row = jnp.full((LANES,), j, jnp.int32)

                def group_body(i, carry, j=j, row=row):
                    acc = list(carry)
                    base = i * (LANES * NACC)
                    for a in range(NACC):
                        off = pl.multiple_of(base + a * LANES, LANES)
                        iv = idx_v[j, pl.ds(off, LANES)]
                        acc[a] = acc[a] + plsc.load_gather(lw_v, [row, iv])
                    return tuple(acc)

                init = (accs[j], zero, zero, zero)
                res = lax.fori_loop(0, GROUPS, group_body, init, unroll=2)
                new_accs.append((res[0] + res[1]) + (res[2] + res[3]))
            return tuple(new_accs)

        issue(0, 0)
        issue(1, 1)

        def triple_body(tp, accs):
            t = 3 * tp
            for ph in range(3):
                drain(t + ph, ph)
                issue(jnp.minimum(t + ph + 2, T - 2), (ph + 2) % 3)
                accs = compute(ph, accs)
            return accs

        zero = jnp.zeros((LANES,), jnp.float32)
        accs = lax.fori_loop(0, (T - 1) // 3, triple_body, (zero,) * BPW)
        drain(T - 2, 0)
        accs = compute(0, accs)
        drain(T - 2, 1)

        out_vec = jnp.zeros((LANES,), jnp.float32)
        for j in range(BPW):
            out_vec = jnp.where(lane == j, jnp.sum(accs[j]), out_vec)
        out_v[...] = out_vec
        pltpu.sync_copy(out_v, gat_hbm.at[wid])

    return body(lw, idx)


def _lse_tc(lw):
    def body(lw_ref, out_ref):
        t = pl.program_id(0)
        x = lw_ref[0]
        m = jnp.max(x, axis=1, keepdims=True)
        s = jnp.sum(jnp.exp(x - m), axis=1)
        lse = m[:, 0] + jnp.log(s)

        @pl.when(t == 0)
        def _():
            out_ref[...] = jnp.zeros_like(out_ref)

        out_ref[0, :] += float(K) * lse

    return pl.pallas_call(
        body,
        grid=(T - 1,),
        in_specs=[pl.BlockSpec((1, B, K), lambda t: (t, 0, 0))],
        out_specs=pl.BlockSpec((1, B), lambda t: (0, 0)),
        out_shape=jax.ShapeDtypeStruct((1, B), jnp.float32),
    )(lw)


def kernel(log_weights, ancestral_indices):
    gat = _gather_sc(log_weights, ancestral_indices)
    lse = _lse_tc(log_weights)
    return gat[:, :BPW].reshape(B) - lse[0]
